# baseline (speedup 1.0000x reference)
import functools

import jax
import jax.numpy as jnp
from jax import lax
from jax.experimental import pallas as pl
from jax.experimental.pallas import tpu as pltpu
from jax.experimental.pallas import tpu_sc as plsc

ROW = 272
BATCH = 16384
NUM_CORES = 2
NUM_SUBCORES = 16
NW = NUM_CORES * NUM_SUBCORES
B_PER_W = BATCH // NW
CHUNK = 128
N_CHUNKS = B_PER_W // CHUNK
NBUF = 2

_mesh = plsc.VectorSubcoreMesh(core_axis_name="c", subcore_axis_name="s")


@functools.partial(
    pl.kernel,
    mesh=_mesh,
    out_type=jax.ShapeDtypeStruct((BATCH, ROW), jnp.bfloat16),
    scratch_types=[
        pltpu.VMEM((N_CHUNKS, CHUNK), jnp.int32),
        pltpu.VMEM((NBUF, CHUNK, ROW), jnp.bfloat16),
        pltpu.SemaphoreType.DMA,
        pltpu.SemaphoreType.DMA,
    ],
    compiler_params=pltpu.CompilerParams(use_tc_tiling_on_sc=False),
)
def _gather_kernel(idx_hbm, table_hbm, out_hbm, idx_v, rows_v, gsem, wsem):
    wid = lax.axis_index("s") * NUM_CORES + lax.axis_index("c")
    base = wid * B_PER_W
    pltpu.sync_copy(idx_hbm.at[wid], idx_v)

    gathers = [None] * N_CHUNKS
    writes = [None] * N_CHUNKS

    def start_gather(c):
        gathers[c] = pltpu.async_copy(
            table_hbm.at[idx_v.at[c]], rows_v.at[c % NBUF], gsem)

    start_gather(0)
    for c in range(N_CHUNKS):
        if c + 1 < N_CHUNKS:
            start_gather(c + 1)
        gathers[c].wait()
        writes[c] = pltpu.async_copy(
            rows_v.at[c % NBUF], out_hbm.at[pl.ds(base + c * CHUNK, CHUNK)],
            wsem)
        if c + NBUF < N_CHUNKS:
            writes[c].wait()
    for c in range(max(0, N_CHUNKS - NBUF), N_CHUNKS):
        writes[c].wait()


def kernel(x_cate, cate_embedding_weight):
    idx = x_cate.astype(jnp.int32).reshape(NW, N_CHUNKS, CHUNK)
    table_bf16 = cate_embedding_weight.astype(jnp.bfloat16)
    return _gather_kernel(idx, table_bf16).astype(jnp.float32)
